# baseline (speedup 1.0000x reference)
import jax
import jax.numpy as jnp
from jax import lax
from jax.experimental import pallas as pl
from jax.experimental.pallas import tpu as pltpu
from jax.experimental.pallas import tpu_sc as plsc

FULL_SHAPE = (100000, 128)
B = FULL_SHAPE[0] * FULL_SHAPE[1]
NUM_W = 1280000
NC = 2
NS = 16
NW = NC * NS
PER_W = B // NW
CH = 8000
SH = 880
SS = CH - SH
NITER = PER_W // CH
NPAIR = NITER // 2
STAGE = NUM_W // NS


def _gather_body(bank_hbm, idx_hbm, out_hbm, bank_sh,
                 idxs0, idxh0, idxs1, idxh1,
                 rows0, rowh0, rows1, rowh1,
                 sem_i, sem_g, sem_h, sem_o):
    cid = lax.axis_index("c")
    sid = lax.axis_index("s")
    wid = sid * NC + cid
    base = wid * PER_W

    def issue_idx(off, i_s, i_h):
        pltpu.async_copy(idx_hbm.at[pl.ds(off, SS)], i_s, sem_i)
        pltpu.async_copy(idx_hbm.at[pl.ds(off + SS, SH)], i_h, sem_i)

    def wait_idx(off, i_s, i_h):
        pltpu.make_async_copy(idx_hbm.at[pl.ds(off, SS)], i_s, sem_i).wait()
        pltpu.make_async_copy(idx_hbm.at[pl.ds(off + SS, SH)], i_h,
                              sem_i).wait()

    def issue_gather(i_s, i_h, r_s, r_h):
        pltpu.async_copy(bank_sh.at[i_s], r_s, sem_g)
        pltpu.async_copy(bank_hbm.at[i_h], r_h, sem_h)

    def wait_gather(i_s, i_h, r_s, r_h):
        pltpu.make_async_copy(bank_sh.at[i_s], r_s, sem_g).wait()
        pltpu.make_async_copy(bank_hbm.at[i_h], r_h, sem_h).wait()

    def issue_store(off, r_s, r_h):
        pltpu.async_copy(r_s, out_hbm.at[pl.ds(off, SS)], sem_o)
        pltpu.async_copy(r_h, out_hbm.at[pl.ds(off + SS, SH)], sem_o)

    def wait_store(off, r_s, r_h):
        pltpu.make_async_copy(r_s, out_hbm.at[pl.ds(off, SS)], sem_o).wait()
        pltpu.make_async_copy(r_h, out_hbm.at[pl.ds(off + SS, SH)],
                              sem_o).wait()

    issue_idx(base, idxs0, idxh0)
    issue_idx(base + CH, idxs1, idxh1)
    pltpu.sync_copy(bank_hbm.at[pl.ds(sid * STAGE, STAGE)],
                    bank_sh.at[pl.ds(sid * STAGE, STAGE)])
    plsc.subcore_barrier()
    wait_idx(base, idxs0, idxh0)
    issue_gather(idxs0, idxh0, rows0, rowh0)

    def step(j, carry):
        off_a = base + (2 * j) * CH
        off_b = off_a + CH

        wait_idx(off_b, idxs1, idxh1)

        @pl.when(j >= 1)
        def _():
            wait_store(off_a, rows1, rowh1)
        issue_gather(idxs1, idxh1, rows1, rowh1)
        wait_gather(idxs0, idxh0, rows0, rowh0)
        issue_store(off_a, rows0, rowh0)

        @pl.when(j + 1 < NPAIR)
        def _():
            issue_idx(off_b + CH, idxs0, idxh0)

        @pl.when(j + 1 < NPAIR)
        def _():
            wait_idx(off_b + CH, idxs0, idxh0)
            wait_store(off_a, rows0, rowh0)
            issue_gather(idxs0, idxh0, rows0, rowh0)
        wait_gather(idxs1, idxh1, rows1, rowh1)
        issue_store(off_b, rows1, rowh1)

        @pl.when(j + 1 < NPAIR)
        def _():
            issue_idx(off_b + 2 * CH, idxs1, idxh1)
        return carry

    lax.fori_loop(0, NPAIR, step, 0)
    wait_store(base, rows0, rowh0)
    wait_store(base, rows1, rowh1)


@jax.jit
def kernel(bank, weight_alloc):
    idx = weight_alloc.reshape(B).astype(jnp.int32)
    call = pl.kernel(
        _gather_body,
        out_type=jax.ShapeDtypeStruct((B,), jnp.float32),
        mesh=plsc.VectorSubcoreMesh(core_axis_name="c", subcore_axis_name="s"),
        scratch_types=[
            pltpu.VMEM_SHARED((NUM_W,), jnp.float32),
            pltpu.VMEM((SS,), jnp.int32),
            pltpu.VMEM((SH,), jnp.int32),
            pltpu.VMEM((SS,), jnp.int32),
            pltpu.VMEM((SH,), jnp.int32),
            pltpu.VMEM((SS,), jnp.float32),
            pltpu.VMEM((SH,), jnp.float32),
            pltpu.VMEM((SS,), jnp.float32),
            pltpu.VMEM((SH,), jnp.float32),
            pltpu.SemaphoreType.DMA,
            pltpu.SemaphoreType.DMA,
            pltpu.SemaphoreType.DMA,
            pltpu.SemaphoreType.DMA,
        ],
    )
    out = call(bank, idx)
    return out.reshape(FULL_SHAPE)


# speedup vs baseline: 1.0413x; 1.0413x over previous
import jax
import jax.numpy as jnp
from jax import lax
from jax.experimental import pallas as pl
from jax.experimental.pallas import tpu as pltpu
from jax.experimental.pallas import tpu_sc as plsc

FULL_SHAPE = (100000, 128)
B = FULL_SHAPE[0] * FULL_SHAPE[1]
NUM_W = 1280000
NC = 2
NS = 16
NW = NC * NS
PER_W = B // NW
CH = 10000
SH = 1096
SS = CH - SH
NITER = PER_W // CH
NPAIR = NITER // 2
STAGE = NUM_W // NS


def _gather_body(bank_hbm, idx_hbm, out_hbm, bank_sh,
                 idxs0, idxh0, idxs1, idxh1,
                 rows0, rowh0, rows1, rowh1,
                 sem_i, sem_g, sem_h, sem_o):
    cid = lax.axis_index("c")
    sid = lax.axis_index("s")
    wid = sid * NC + cid
    base = wid * PER_W

    def issue_idx(off, i_s, i_h):
        pltpu.async_copy(idx_hbm.at[pl.ds(off, SS)], i_s, sem_i)
        pltpu.async_copy(idx_hbm.at[pl.ds(off + SS, SH)], i_h, sem_i)

    def wait_idx(off, i_s, i_h):
        pltpu.make_async_copy(idx_hbm.at[pl.ds(off, SS)], i_s, sem_i).wait()
        pltpu.make_async_copy(idx_hbm.at[pl.ds(off + SS, SH)], i_h,
                              sem_i).wait()

    def issue_gather(i_s, i_h, r_s, r_h):
        pltpu.async_copy(bank_sh.at[i_s], r_s, sem_g)
        pltpu.async_copy(bank_hbm.at[i_h], r_h, sem_h)

    def wait_gather(i_s, i_h, r_s, r_h):
        pltpu.make_async_copy(bank_sh.at[i_s], r_s, sem_g).wait()
        pltpu.make_async_copy(bank_hbm.at[i_h], r_h, sem_h).wait()

    def issue_store(off, r_s, r_h):
        pltpu.async_copy(r_s, out_hbm.at[pl.ds(off, SS)], sem_o)
        pltpu.async_copy(r_h, out_hbm.at[pl.ds(off + SS, SH)], sem_o)

    def wait_store(off, r_s, r_h):
        pltpu.make_async_copy(r_s, out_hbm.at[pl.ds(off, SS)], sem_o).wait()
        pltpu.make_async_copy(r_h, out_hbm.at[pl.ds(off + SS, SH)],
                              sem_o).wait()

    issue_idx(base, idxs0, idxh0)
    issue_idx(base + CH, idxs1, idxh1)
    pltpu.sync_copy(bank_hbm.at[pl.ds(sid * STAGE, STAGE)],
                    bank_sh.at[pl.ds(sid * STAGE, STAGE)])
    plsc.subcore_barrier()
    wait_idx(base, idxs0, idxh0)
    issue_gather(idxs0, idxh0, rows0, rowh0)

    def step(j, carry):
        off_a = base + (2 * j) * CH
        off_b = off_a + CH

        wait_idx(off_b, idxs1, idxh1)

        @pl.when(j >= 1)
        def _():
            wait_store(off_a, rows1, rowh1)
        issue_gather(idxs1, idxh1, rows1, rowh1)
        wait_gather(idxs0, idxh0, rows0, rowh0)
        issue_store(off_a, rows0, rowh0)

        @pl.when(j + 1 < NPAIR)
        def _():
            issue_idx(off_b + CH, idxs0, idxh0)

        @pl.when(j + 1 < NPAIR)
        def _():
            wait_idx(off_b + CH, idxs0, idxh0)
            wait_store(off_a, rows0, rowh0)
            issue_gather(idxs0, idxh0, rows0, rowh0)
        wait_gather(idxs1, idxh1, rows1, rowh1)
        issue_store(off_b, rows1, rowh1)

        @pl.when(j + 1 < NPAIR)
        def _():
            issue_idx(off_b + 2 * CH, idxs1, idxh1)
        return carry

    lax.fori_loop(0, NPAIR, step, 0)
    wait_store(base, rows0, rowh0)
    wait_store(base, rows1, rowh1)


@jax.jit
def kernel(bank, weight_alloc):
    idx = weight_alloc.reshape(B).astype(jnp.int32)
    call = pl.kernel(
        _gather_body,
        out_type=jax.ShapeDtypeStruct((B,), jnp.float32),
        mesh=plsc.VectorSubcoreMesh(core_axis_name="c", subcore_axis_name="s"),
        scratch_types=[
            pltpu.VMEM_SHARED((NUM_W,), jnp.float32),
            pltpu.VMEM((SS,), jnp.int32),
            pltpu.VMEM((SH,), jnp.int32),
            pltpu.VMEM((SS,), jnp.int32),
            pltpu.VMEM((SH,), jnp.int32),
            pltpu.VMEM((SS,), jnp.float32),
            pltpu.VMEM((SH,), jnp.float32),
            pltpu.VMEM((SS,), jnp.float32),
            pltpu.VMEM((SH,), jnp.float32),
            pltpu.SemaphoreType.DMA,
            pltpu.SemaphoreType.DMA,
            pltpu.SemaphoreType.DMA,
            pltpu.SemaphoreType.DMA,
        ],
    )
    out = call(bank, idx)
    return out.reshape(FULL_SHAPE)
